# baseline (speedup 1.0000x reference)
import math

import jax
import jax.numpy as jnp
from jax.experimental import pallas as pl
from jax.experimental.pallas import tpu as pltpu

_ORIENTATIONS = 10
_PPC = 8
_CPB = 2
_MAX_ANGLE = math.pi
_EPS = 1e-5

_H = 512
_W = 512
_HC = _H // _PPC
_WC = _W // _PPC
_HN = _HC - _CPB + 1
_WN = _WC - _CPB + 1
_FW = _WN * _CPB * _CPB


def _hog_body(img_ref, pool_ref, exp_ref, out_ref, pad_ref, rp_ref):
    @pl.when(pl.program_id(0) == 0)
    def _():
        pad_ref[0:1, :] = jnp.zeros((1, _W + 2), jnp.float32)
        pad_ref[_H + 1:_H + 2, :] = jnp.zeros((1, _W + 2), jnp.float32)
        pad_ref[:, 0:1] = jnp.zeros((_H + 2, 1), jnp.float32)
        pad_ref[:, _W + 1:_W + 2] = jnp.zeros((_H + 2, 1), jnp.float32)

    pad_ref[1:_H + 1, 1:_W + 1] = (
        img_ref[0, 0].astype(jnp.bfloat16).astype(jnp.float32))
    n_slabs = _H // 64
    for s in range(n_slabs):
        p = pad_ref[64 * s:64 * s + 66, :]

        def tap(dy, dx):
            return p[dy:dy + 64, dx:dx + _W]

        gx = ((((tap(0, 0) - tap(0, 2)) + 2.0 * tap(1, 0))
               - 2.0 * tap(1, 2)) + tap(2, 0)) - tap(2, 2)
        gy = ((((tap(0, 0) + 2.0 * tap(0, 1)) + tap(0, 2))
               - tap(2, 0)) - 2.0 * tap(2, 1)) - tap(2, 2)

        mag = jnp.sqrt(gx * gx + gy * gy)
        t = jnp.arctan2(gx, gy) / _MAX_ANGLE * _ORIENTATIONS
        f = jnp.floor(t)
        is_int = t == f
        fi = f.astype(jnp.int32)
        fo = jnp.where(fi < 0, fi + _ORIENTATIONS, fi)
        fo = jnp.where(fo == _ORIENTATIONS, 0, fo)
        one_minus = 1.0 - mag
        a = jnp.where(is_int, mag + one_minus, mag)
        b = jnp.where(is_int, 0.0, one_minus)

        zero = jnp.zeros_like(mag)
        prev = fo == (_ORIENTATIONS - 1)
        for o in range(_ORIENTATIONS):
            m = fo == o
            contrib = jnp.where(m, a, zero) + jnp.where(prev, b, zero)
            rp_ref[64 * o + 8 * s:64 * o + 8 * s + 8, :] = (
                contrib.reshape(8, _PPC, _W).sum(axis=1))
            prev = m

    pooled = jax.lax.dot(rp_ref[...], pool_ref[...],
                         precision=jax.lax.Precision.HIGHEST)
    h = pooled.reshape(_ORIENTATIONS, _HC, _WC) * (1.0 / (_PPC * _PPC))

    h00 = h[:, 0:_HN, 0:_WN]
    h01 = h[:, 0:_HN, 1:_WC]
    h10 = h[:, 1:_HC, 0:_WN]
    h11 = h[:, 1:_HC, 1:_WC]
    blk = h00 + h01 + h10 + h11
    inv = 1.0 / jnp.sqrt(blk * blk + _EPS * _EPS)
    y00 = h00 * inv
    y01 = h01 * inv
    y10 = h10 * inv
    y11 = h11 * inv
    ys = [y00, y01, y10, y11]
    his = [y.astype(jnp.bfloat16).astype(jnp.float32) for y in ys]
    los = [y - hi for y, hi in zip(ys, his)]
    for o in range(_ORIENTATIONS):
        acc = jax.lax.dot(his[0][o], exp_ref[0])
        for q in range(1, _CPB * _CPB):
            acc += jax.lax.dot(his[q][o], exp_ref[q])
        for q in range(_CPB * _CPB):
            acc += jax.lax.dot(los[q][o], exp_ref[q])
        out_ref[0, o] = acc


@jax.jit
def kernel(img):
    n = img.shape[0]
    pool = (jnp.arange(_W)[:, None] // _PPC
            == jnp.arange(_WC)[None, :]).astype(jnp.float32)
    exp = (jnp.arange(_FW)[None, None, :]
           == 4 * jnp.arange(_WN)[None, :, None]
           + jnp.arange(_CPB * _CPB)[:, None, None]).astype(jnp.float32)

    out = pl.pallas_call(
        _hog_body,
        grid=(n,),
        in_specs=[
            pl.BlockSpec((1, 1, _H, _W), lambda i: (i, 0, 0, 0)),
            pl.BlockSpec((_W, _WC), lambda i: (0, 0)),
            pl.BlockSpec((_CPB * _CPB, _WN, _FW), lambda i: (0, 0, 0)),
        ],
        out_specs=pl.BlockSpec((1, _ORIENTATIONS, _HN, _FW),
                               lambda i: (i, 0, 0, 0)),
        out_shape=jax.ShapeDtypeStruct(
            (n, _ORIENTATIONS, _HN, _FW), jnp.float32),
        scratch_shapes=[pltpu.VMEM((_H + 2, _W + 2), jnp.float32),
                        pltpu.VMEM((64 * _ORIENTATIONS, _W), jnp.float32)],
        compiler_params=pltpu.CompilerParams(
            dimension_semantics=("arbitrary",)),
    )(img, pool, exp)

    return out.reshape(n, _ORIENTATIONS * _HN * _FW)
